# Initial kernel scaffold; baseline (speedup 1.0000x reference)
import jax
import jax.numpy as jnp
from jax.experimental import pallas as pl


def kernel(user_emb, item_emb, h_list, t_list):
    raise NotImplementedError("write your pallas kernel here")


# speedup vs baseline: 11.6543x; 11.6543x over previous
import functools

import jax
import jax.numpy as jnp
from jax import lax
from jax.experimental import pallas as pl
from jax.experimental.pallas import tpu as pltpu
from jax.experimental.pallas import tpu_sc as plsc

N_USERS = 5000
N_ITEMS = 5000
N = N_USERS + N_ITEMS
E = 320000
D = 128
NC = 2
NS = 16
NW = NC * NS
EPW = E // NW
CH = 80
SCH = 25
SCE = SCH * CH
NSUP = EPW // SCE
NSC = NW * NSUP
NPAD = 10240
RPT = NPAD // NW
SPT = NPAD // NS
APT = NPAD // NS
AZC = 80
CR = 80
G16 = 16

_mesh = plsc.VectorSubcoreMesh(core_axis_name="c", subcore_axis_name="s")
_params = pltpu.CompilerParams(needs_layout_passes=False)


def _rsqrt16(x):
    i = lax.bitcast_convert_type(x, jnp.int32)
    i = jnp.int32(0x5F3759DF) - (i >> 1)
    y = lax.bitcast_convert_type(i, jnp.float32)
    for _ in range(3):
        y = y * (1.5 - 0.5 * x * y * y)
    return y


@functools.partial(
    pl.kernel,
    out_type=jax.ShapeDtypeStruct((NC, NPAD), jnp.float32),
    mesh=_mesh,
    compiler_params=_params,
    scratch_types=[
        pltpu.VMEM((SCH, CH), jnp.int32),
        pltpu.VMEM((CH,), jnp.float32),
        pltpu.VMEM((SPT,), jnp.float32),
        pltpu.VMEM_SHARED((NPAD,), jnp.float32),
    ],
)
def _deg_kernel(h3_hbm, degp_hbm, h3s, ones_v, z_v, deg_sh):
    cid = lax.axis_index("c")
    sid = lax.axis_index("s")
    wid = sid * NC + cid

    def fill_ones(i, c):
        ones_v[pl.ds(i * G16, G16)] = jnp.full((G16,), 1.0, jnp.float32)
        return c

    lax.fori_loop(0, CH // G16, fill_ones, 0)

    def fill_zero(i, c):
        z_v[pl.ds(i * G16, G16)] = jnp.zeros((G16,), jnp.float32)
        return c

    lax.fori_loop(0, SPT // G16, fill_zero, 0)
    pltpu.sync_copy(z_v, deg_sh.at[pl.ds(sid * SPT, SPT)])
    plsc.subcore_barrier()

    for s in range(NSUP):
        pltpu.sync_copy(h3_hbm.at[wid * NSUP + s], h3s)

        def scat(j, c):
            pltpu.sync_copy(ones_v, deg_sh.at[h3s.at[j]], add=True)
            return c

        lax.fori_loop(0, SCH, scat, 0)
    plsc.subcore_barrier()
    pltpu.sync_copy(deg_sh.at[pl.ds(sid * SPT, SPT)], z_v)
    pltpu.sync_copy(z_v, degp_hbm.at[cid, pl.ds(sid * SPT, SPT)])


def _zero_acc(buf, acc_sh, sid):
    def zrow(r, c):
        for k in range(D // G16):
            buf[r, pl.ds(k * G16, G16)] = jnp.zeros((G16,), jnp.float32)
        return c

    lax.fori_loop(0, CH, zrow, 0)
    for i in range(APT // AZC):
        pltpu.sync_copy(buf, acc_sh.at[pl.ds(sid * APT + i * AZC, AZC)])


def _scale_rows(buf, g_v, j):
    base = j * CH

    def blk(q, c):
        gvec = g_v[pl.ds(base + q * G16, G16)]
        for r16 in range(G16):
            gb = jnp.full((G16,), gvec[r16], jnp.float32)
            row = q * G16 + r16
            for k in range(D // G16):
                buf[row, pl.ds(k * G16, G16)] = buf[row, pl.ds(k * G16, G16)] * gb
        return c

    lax.fori_loop(0, CH // G16, blk, 0)


def _spmm_super(x_hbm, h3s, t1s, g_v, buf, acc_sh, gsem):
    def mainbody(j, c):
        pltpu.async_copy(x_hbm.at[t1s.at[pl.ds(j * CH, CH)]], buf, gsem).wait()
        _scale_rows(buf, g_v, j)
        pltpu.sync_copy(buf, acc_sh.at[h3s.at[j]], add=True)
        return c

    lax.fori_loop(0, SCH, mainbody, 0)


def _write_partial(acc_sh, part_hbm, cid, sid):
    for i in range(APT // AZC):
        rows = pl.ds(sid * APT + i * AZC, AZC)
        pltpu.sync_copy(acc_sh.at[rows], part_hbm.at[cid, rows])


@functools.partial(
    pl.kernel,
    out_type=(
        jax.ShapeDtypeStruct((NC, NPAD, D), jnp.float32),
        jax.ShapeDtypeStruct((NSC, SCE), jnp.float32),
    ),
    mesh=_mesh,
    compiler_params=_params,
    scratch_types=[
        pltpu.VMEM((SCH, CH), jnp.int32),
        pltpu.VMEM((SCE,), jnp.int32),
        pltpu.VMEM((SCE,), jnp.int32),
        pltpu.VMEM((SCE,), jnp.float32),
        pltpu.VMEM((NPAD,), jnp.float32),
        pltpu.VMEM((SPT,), jnp.float32),
        pltpu.VMEM((CH, D), jnp.float32),
        pltpu.VMEM_SHARED((NPAD, D), jnp.float32),
        pltpu.SemaphoreType.DMA,
    ],
)
def _layer1_kernel(x_hbm, h3_hbm, hf_hbm, tf_hbm, degp_hbm, part_hbm, g_hbm,
                   h3s, h1s, t1s, g_v, dis_v, dtmp, buf, acc_sh, gsem):
    cid = lax.axis_index("c")
    sid = lax.axis_index("s")
    wid = sid * NC + cid
    _zero_acc(buf, acc_sh, sid)

    pltpu.sync_copy(degp_hbm.at[0], dis_v)
    for p in range(NPAD // SPT):
        pltpu.sync_copy(degp_hbm.at[1, pl.ds(p * SPT, SPT)], dtmp)

        def disbody(i, c):
            sl = pl.ds(p * SPT + i * G16, G16)
            d = dis_v[sl] + dtmp[pl.ds(i * G16, G16)]
            r = _rsqrt16(jnp.maximum(d, 1.0))
            dis_v[sl] = jnp.where(d > 0.0, r, 0.0)
            return c

        lax.fori_loop(0, SPT // G16, disbody, 0)

    plsc.subcore_barrier()
    for s in range(NSUP):
        sc = wid * NSUP + s
        pltpu.sync_copy(h3_hbm.at[sc], h3s)
        pltpu.sync_copy(hf_hbm.at[sc], h1s)
        pltpu.sync_copy(tf_hbm.at[sc], t1s)

        def gbody(i, c):
            sl = pl.ds(i * G16, G16)
            gh = plsc.load_gather(dis_v, [h1s[sl]])
            gt = plsc.load_gather(dis_v, [t1s[sl]])
            g_v[sl] = gh * gt
            return c

        lax.fori_loop(0, SCE // G16, gbody, 0)
        pltpu.sync_copy(g_v, g_hbm.at[sc])
        _spmm_super(x_hbm, h3s, t1s, g_v, buf, acc_sh, gsem)
    plsc.subcore_barrier()
    _write_partial(acc_sh, part_hbm, cid, sid)


@functools.partial(
    pl.kernel,
    out_type=jax.ShapeDtypeStruct((NC, NPAD, D), jnp.float32),
    mesh=_mesh,
    compiler_params=_params,
    scratch_types=[
        pltpu.VMEM((SCH, CH), jnp.int32),
        pltpu.VMEM((SCE,), jnp.int32),
        pltpu.VMEM((SCE,), jnp.float32),
        pltpu.VMEM((CH, D), jnp.float32),
        pltpu.VMEM_SHARED((NPAD, D), jnp.float32),
        pltpu.SemaphoreType.DMA,
    ],
)
def _layer2_kernel(x_hbm, h3_hbm, tf_hbm, g_hbm, part_hbm,
                   h3s, t1s, g_v, buf, acc_sh, gsem):
    cid = lax.axis_index("c")
    sid = lax.axis_index("s")
    wid = sid * NC + cid
    _zero_acc(buf, acc_sh, sid)
    plsc.subcore_barrier()
    for s in range(NSUP):
        sc = wid * NSUP + s
        pltpu.sync_copy(h3_hbm.at[sc], h3s)
        pltpu.sync_copy(tf_hbm.at[sc], t1s)
        pltpu.sync_copy(g_hbm.at[sc], g_v)
        _spmm_super(x_hbm, h3s, t1s, g_v, buf, acc_sh, gsem)
    plsc.subcore_barrier()
    _write_partial(acc_sh, part_hbm, cid, sid)


@functools.partial(
    pl.kernel,
    out_type=(
        jax.ShapeDtypeStruct((N, D), jnp.float32),
        jax.ShapeDtypeStruct((N, D), jnp.float32),
    ),
    mesh=_mesh,
    compiler_params=_params,
    scratch_types=[
        pltpu.VMEM((CR, D), jnp.float32),
        pltpu.VMEM((CR, D), jnp.float32),
        pltpu.VMEM((CR, D), jnp.float32),
    ],
)
def _combine1_kernel(part_hbm, x0_hbm, out1_hbm, emb1_hbm, pa, pb, px):
    cid = lax.axis_index("c")
    sid = lax.axis_index("s")
    wid = sid * NC + cid
    base = pl.multiple_of(jnp.minimum(wid * RPT, N - RPT), 16)
    for i in range(RPT // CR):
        b = base + i * CR
        pltpu.sync_copy(part_hbm.at[0, pl.ds(b, CR)], pa)
        pltpu.sync_copy(part_hbm.at[1, pl.ds(b, CR)], pb)
        pltpu.sync_copy(x0_hbm.at[pl.ds(b, CR)], px)

        def rb(r, c):
            for k in range(D // G16):
                sl = pl.ds(k * G16, G16)
                sm = pa[r, sl] + pb[r, sl]
                pa[r, sl] = sm
                px[r, sl] = sm + px[r, sl]
            return c

        lax.fori_loop(0, CR, rb, 0)
        pltpu.sync_copy(pa, out1_hbm.at[pl.ds(b, CR)])
        pltpu.sync_copy(px, emb1_hbm.at[pl.ds(b, CR)])


@functools.partial(
    pl.kernel,
    out_type=(
        jax.ShapeDtypeStruct((N, D), jnp.float32),
        jax.ShapeDtypeStruct((N, D), jnp.float32),
    ),
    mesh=_mesh,
    compiler_params=_params,
    scratch_types=[
        pltpu.VMEM((CR, D), jnp.float32),
        pltpu.VMEM((CR, D), jnp.float32),
        pltpu.VMEM((CR, D), jnp.float32),
        pltpu.VMEM((CR, D), jnp.float32),
    ],
)
def _combine2_kernel(part_hbm, x0_hbm, emb1_hbm, out2_hbm, summed_hbm,
                     pa, pb, px, pe):
    cid = lax.axis_index("c")
    sid = lax.axis_index("s")
    wid = sid * NC + cid
    base = pl.multiple_of(jnp.minimum(wid * RPT, N - RPT), 16)
    for i in range(RPT // CR):
        b = base + i * CR
        pltpu.sync_copy(part_hbm.at[0, pl.ds(b, CR)], pa)
        pltpu.sync_copy(part_hbm.at[1, pl.ds(b, CR)], pb)
        pltpu.sync_copy(x0_hbm.at[pl.ds(b, CR)], px)
        pltpu.sync_copy(emb1_hbm.at[pl.ds(b, CR)], pe)

        def rb(r, c):
            for k in range(D // G16):
                sl = pl.ds(k * G16, G16)
                o2 = pa[r, sl] + pb[r, sl]
                pa[r, sl] = o2
                pe[r, sl] = px[r, sl] + 2.0 * pe[r, sl] + o2
            return c

        lax.fori_loop(0, CR, rb, 0)
        pltpu.sync_copy(pa, out2_hbm.at[pl.ds(b, CR)])
        pltpu.sync_copy(pe, summed_hbm.at[pl.ds(b, CR)])


def kernel(user_emb, item_emb, h_list, t_list):
    x0 = jnp.concatenate([user_emb, item_emb], axis=0)
    h3 = h_list.reshape(NSC, SCH, CH)
    hf = h_list.reshape(NSC, SCE)
    tf = t_list.reshape(NSC, SCE)
    degp = _deg_kernel(h3)
    part1, g = _layer1_kernel(x0, h3, hf, tf, degp)
    out1, emb1 = _combine1_kernel(part1, x0)
    part2 = _layer2_kernel(emb1, h3, tf, g)
    out2, summed = _combine2_kernel(part2, x0, emb1)
    return summed[:N_USERS], summed[N_USERS:], out1, out2


# speedup vs baseline: 13.4532x; 1.1544x over previous
import functools

import jax
import jax.numpy as jnp
from jax import lax
from jax.experimental import pallas as pl
from jax.experimental.pallas import tpu as pltpu
from jax.experimental.pallas import tpu_sc as plsc

N_USERS = 5000
N_ITEMS = 5000
N = N_USERS + N_ITEMS
E = 320000
D = 128
NC = 2
NS = 16
NW = NC * NS
EPW = E // NW
CH = 80
SCH = 25
SCE = SCH * CH
NSUP = EPW // SCE
NSC = NW * NSUP
NPAD = 10240
RPT = NPAD // NW
SPT = NPAD // NS
APT = NPAD // NS
AZC = 80
CR = 80
G16 = 16

_mesh = plsc.VectorSubcoreMesh(core_axis_name="c", subcore_axis_name="s")
_params = pltpu.CompilerParams(needs_layout_passes=False)


def _rsqrt16(x):
    i = lax.bitcast_convert_type(x, jnp.int32)
    i = jnp.int32(0x5F3759DF) - (i >> 1)
    y = lax.bitcast_convert_type(i, jnp.float32)
    for _ in range(3):
        y = y * (1.5 - 0.5 * x * y * y)
    return y


@functools.partial(
    pl.kernel,
    out_type=jax.ShapeDtypeStruct((NC, NPAD), jnp.float32),
    mesh=_mesh,
    compiler_params=_params,
    scratch_types=[
        pltpu.VMEM((SCH, CH), jnp.int32),
        pltpu.VMEM((CH,), jnp.float32),
        pltpu.VMEM((SPT,), jnp.float32),
        pltpu.VMEM_SHARED((NPAD,), jnp.float32),
    ],
)
def _deg_kernel(h3_hbm, degp_hbm, h3s, ones_v, z_v, deg_sh):
    cid = lax.axis_index("c")
    sid = lax.axis_index("s")
    wid = sid * NC + cid

    def fill_ones(i, c):
        ones_v[pl.ds(i * G16, G16)] = jnp.full((G16,), 1.0, jnp.float32)
        return c

    lax.fori_loop(0, CH // G16, fill_ones, 0)

    def fill_zero(i, c):
        z_v[pl.ds(i * G16, G16)] = jnp.zeros((G16,), jnp.float32)
        return c

    lax.fori_loop(0, SPT // G16, fill_zero, 0)
    pltpu.sync_copy(z_v, deg_sh.at[pl.ds(sid * SPT, SPT)])
    plsc.subcore_barrier()

    for s in range(NSUP):
        pltpu.sync_copy(h3_hbm.at[wid * NSUP + s], h3s)

        def scat(j, c):
            pltpu.sync_copy(ones_v, deg_sh.at[h3s.at[j]], add=True)
            return c

        lax.fori_loop(0, SCH, scat, 0)
    plsc.subcore_barrier()
    pltpu.sync_copy(deg_sh.at[pl.ds(sid * SPT, SPT)], z_v)
    pltpu.sync_copy(z_v, degp_hbm.at[cid, pl.ds(sid * SPT, SPT)])


def _zero_acc(buf, acc_sh, sid):
    def zrow(r, c):
        for k in range(D // G16):
            buf[r, pl.ds(k * G16, G16)] = jnp.zeros((G16,), jnp.float32)
        return c

    lax.fori_loop(0, CH, zrow, 0)
    for i in range(APT // AZC):
        pltpu.sync_copy(buf, acc_sh.at[pl.ds(sid * APT + i * AZC, AZC)])


def _scale_rows(buf, g_v, j):
    base = j * CH

    def blk(q, c):
        gvec = g_v[pl.ds(base + q * G16, G16)]
        for r16 in range(G16):
            gb = jnp.full((G16,), gvec[r16], jnp.float32)
            row = q * G16 + r16
            for k in range(D // G16):
                buf[row, pl.ds(k * G16, G16)] = buf[row, pl.ds(k * G16, G16)] * gb
        return c

    lax.fori_loop(0, CH // G16, blk, 0)


def _zero_buf(buf):
    def zrow(r, c):
        for k in range(D // G16):
            buf[r, pl.ds(k * G16, G16)] = jnp.zeros((G16,), jnp.float32)
        return c

    lax.fori_loop(0, CH, zrow, 0)


def _spmm_super(x_hbm, h3s, t1s, g_v, bufa, bufb, acc_sh, gsa, gsb, ssa, ssb):
    def gather(j, buf, sem):
        off = pl.multiple_of(j * CH, 16)
        return pltpu.async_copy(x_hbm.at[t1s.at[pl.ds(off, CH)]], buf, sem)

    def gwait(buf, sem):
        pltpu.make_async_copy(x_hbm.at[t1s.at[pl.ds(0, CH)]], buf, sem).wait()

    def scat(j, buf, sem):
        return pltpu.async_copy(buf, acc_sh.at[h3s.at[j]], sem, add=True)

    def swait(buf, sem):
        pltpu.make_async_copy(buf, acc_sh.at[h3s.at[0]], sem).wait()

    _zero_buf(bufb)
    scat(0, bufb, ssb)
    gather(0, bufa, gsa)

    def pairbody(jj, c):
        j0 = jj * 2
        j1 = j0 + 1
        gwait(bufa, gsa)
        _scale_rows(bufa, g_v, j0)
        swait(bufb, ssb)
        gather(j1, bufb, gsb)
        scat(j0, bufa, ssa)
        gwait(bufb, gsb)
        _scale_rows(bufb, g_v, j1)
        swait(bufa, ssa)
        gather(j1 + 1, bufa, gsa)
        scat(j1, bufb, ssb)
        return c

    lax.fori_loop(0, (SCH - 1) // 2, pairbody, 0)
    gwait(bufa, gsa)
    _scale_rows(bufa, g_v, SCH - 1)
    swait(bufb, ssb)
    scat(SCH - 1, bufa, ssa)
    swait(bufa, ssa)


def _write_partial(acc_sh, part_hbm, cid, sid):
    for i in range(APT // AZC):
        rows = pl.ds(sid * APT + i * AZC, AZC)
        pltpu.sync_copy(acc_sh.at[rows], part_hbm.at[cid, rows])


@functools.partial(
    pl.kernel,
    out_type=(
        jax.ShapeDtypeStruct((NC, NPAD, D), jnp.float32),
        jax.ShapeDtypeStruct((NSC, SCE), jnp.float32),
    ),
    mesh=_mesh,
    compiler_params=_params,
    scratch_types=[
        pltpu.VMEM((SCH, CH), jnp.int32),
        pltpu.VMEM((SCE,), jnp.int32),
        pltpu.VMEM((SCE,), jnp.int32),
        pltpu.VMEM((SCE,), jnp.float32),
        pltpu.VMEM((NPAD,), jnp.float32),
        pltpu.VMEM((SPT,), jnp.float32),
        pltpu.VMEM((CH, D), jnp.float32),
        pltpu.VMEM((CH, D), jnp.float32),
        pltpu.VMEM_SHARED((NPAD, D), jnp.float32),
        pltpu.SemaphoreType.DMA,
        pltpu.SemaphoreType.DMA,
        pltpu.SemaphoreType.DMA,
        pltpu.SemaphoreType.DMA,
    ],
)
def _layer1_kernel(x_hbm, h3_hbm, hf_hbm, tf_hbm, degp_hbm, part_hbm, g_hbm,
                   h3s, h1s, t1s, g_v, dis_v, dtmp, bufa, bufb, acc_sh,
                   gsa, gsb, ssa, ssb):
    cid = lax.axis_index("c")
    sid = lax.axis_index("s")
    wid = sid * NC + cid
    _zero_acc(bufa, acc_sh, sid)

    pltpu.sync_copy(degp_hbm.at[0], dis_v)
    for p in range(NPAD // SPT):
        pltpu.sync_copy(degp_hbm.at[1, pl.ds(p * SPT, SPT)], dtmp)

        def disbody(i, c):
            sl = pl.ds(p * SPT + i * G16, G16)
            d = dis_v[sl] + dtmp[pl.ds(i * G16, G16)]
            r = _rsqrt16(jnp.maximum(d, 1.0))
            dis_v[sl] = jnp.where(d > 0.0, r, 0.0)
            return c

        lax.fori_loop(0, SPT // G16, disbody, 0)

    plsc.subcore_barrier()
    for s in range(NSUP):
        sc = wid * NSUP + s
        pltpu.sync_copy(h3_hbm.at[sc], h3s)
        pltpu.sync_copy(hf_hbm.at[sc], h1s)
        pltpu.sync_copy(tf_hbm.at[sc], t1s)

        def gbody(i, c):
            sl = pl.ds(i * G16, G16)
            gh = plsc.load_gather(dis_v, [h1s[sl]])
            gt = plsc.load_gather(dis_v, [t1s[sl]])
            g_v[sl] = gh * gt
            return c

        lax.fori_loop(0, SCE // G16, gbody, 0)
        pltpu.sync_copy(g_v, g_hbm.at[sc])
        _spmm_super(x_hbm, h3s, t1s, g_v, bufa, bufb, acc_sh,
                    gsa, gsb, ssa, ssb)
    plsc.subcore_barrier()
    _write_partial(acc_sh, part_hbm, cid, sid)


@functools.partial(
    pl.kernel,
    out_type=jax.ShapeDtypeStruct((NC, NPAD, D), jnp.float32),
    mesh=_mesh,
    compiler_params=_params,
    scratch_types=[
        pltpu.VMEM((SCH, CH), jnp.int32),
        pltpu.VMEM((SCE,), jnp.int32),
        pltpu.VMEM((SCE,), jnp.float32),
        pltpu.VMEM((CH, D), jnp.float32),
        pltpu.VMEM((CH, D), jnp.float32),
        pltpu.VMEM_SHARED((NPAD, D), jnp.float32),
        pltpu.SemaphoreType.DMA,
        pltpu.SemaphoreType.DMA,
        pltpu.SemaphoreType.DMA,
        pltpu.SemaphoreType.DMA,
    ],
)
def _layer2_kernel(x_hbm, h3_hbm, tf_hbm, g_hbm, part_hbm,
                   h3s, t1s, g_v, bufa, bufb, acc_sh, gsa, gsb, ssa, ssb):
    cid = lax.axis_index("c")
    sid = lax.axis_index("s")
    wid = sid * NC + cid
    _zero_acc(bufa, acc_sh, sid)
    plsc.subcore_barrier()
    for s in range(NSUP):
        sc = wid * NSUP + s
        pltpu.sync_copy(h3_hbm.at[sc], h3s)
        pltpu.sync_copy(tf_hbm.at[sc], t1s)
        pltpu.sync_copy(g_hbm.at[sc], g_v)
        _spmm_super(x_hbm, h3s, t1s, g_v, bufa, bufb, acc_sh,
                    gsa, gsb, ssa, ssb)
    plsc.subcore_barrier()
    _write_partial(acc_sh, part_hbm, cid, sid)


@functools.partial(
    pl.kernel,
    out_type=(
        jax.ShapeDtypeStruct((N, D), jnp.float32),
        jax.ShapeDtypeStruct((N, D), jnp.float32),
    ),
    mesh=_mesh,
    compiler_params=_params,
    scratch_types=[
        pltpu.VMEM((CR, D), jnp.float32),
        pltpu.VMEM((CR, D), jnp.float32),
        pltpu.VMEM((CR, D), jnp.float32),
    ],
)
def _combine1_kernel(part_hbm, x0_hbm, out1_hbm, emb1_hbm, pa, pb, px):
    cid = lax.axis_index("c")
    sid = lax.axis_index("s")
    wid = sid * NC + cid
    base = pl.multiple_of(jnp.minimum(wid * RPT, N - RPT), 16)
    for i in range(RPT // CR):
        b = base + i * CR
        pltpu.sync_copy(part_hbm.at[0, pl.ds(b, CR)], pa)
        pltpu.sync_copy(part_hbm.at[1, pl.ds(b, CR)], pb)
        pltpu.sync_copy(x0_hbm.at[pl.ds(b, CR)], px)

        def rb(r, c):
            for k in range(D // G16):
                sl = pl.ds(k * G16, G16)
                sm = pa[r, sl] + pb[r, sl]
                pa[r, sl] = sm
                px[r, sl] = sm + px[r, sl]
            return c

        lax.fori_loop(0, CR, rb, 0)
        pltpu.sync_copy(pa, out1_hbm.at[pl.ds(b, CR)])
        pltpu.sync_copy(px, emb1_hbm.at[pl.ds(b, CR)])


@functools.partial(
    pl.kernel,
    out_type=(
        jax.ShapeDtypeStruct((N, D), jnp.float32),
        jax.ShapeDtypeStruct((N, D), jnp.float32),
    ),
    mesh=_mesh,
    compiler_params=_params,
    scratch_types=[
        pltpu.VMEM((CR, D), jnp.float32),
        pltpu.VMEM((CR, D), jnp.float32),
        pltpu.VMEM((CR, D), jnp.float32),
        pltpu.VMEM((CR, D), jnp.float32),
    ],
)
def _combine2_kernel(part_hbm, x0_hbm, emb1_hbm, out2_hbm, summed_hbm,
                     pa, pb, px, pe):
    cid = lax.axis_index("c")
    sid = lax.axis_index("s")
    wid = sid * NC + cid
    base = pl.multiple_of(jnp.minimum(wid * RPT, N - RPT), 16)
    for i in range(RPT // CR):
        b = base + i * CR
        pltpu.sync_copy(part_hbm.at[0, pl.ds(b, CR)], pa)
        pltpu.sync_copy(part_hbm.at[1, pl.ds(b, CR)], pb)
        pltpu.sync_copy(x0_hbm.at[pl.ds(b, CR)], px)
        pltpu.sync_copy(emb1_hbm.at[pl.ds(b, CR)], pe)

        def rb(r, c):
            for k in range(D // G16):
                sl = pl.ds(k * G16, G16)
                o2 = pa[r, sl] + pb[r, sl]
                pa[r, sl] = o2
                pe[r, sl] = px[r, sl] + 2.0 * pe[r, sl] + o2
            return c

        lax.fori_loop(0, CR, rb, 0)
        pltpu.sync_copy(pa, out2_hbm.at[pl.ds(b, CR)])
        pltpu.sync_copy(pe, summed_hbm.at[pl.ds(b, CR)])


def kernel(user_emb, item_emb, h_list, t_list):
    x0 = jnp.concatenate([user_emb, item_emb], axis=0)
    h3 = h_list.reshape(NSC, SCH, CH)
    hf = h_list.reshape(NSC, SCE)
    tf = t_list.reshape(NSC, SCE)
    degp = _deg_kernel(h3)
    part1, g = _layer1_kernel(x0, h3, hf, tf, degp)
    out1, emb1 = _combine1_kernel(part1, x0)
    part2 = _layer2_kernel(emb1, h3, tf, g)
    out2, summed = _combine2_kernel(part2, x0, emb1)
    return summed[:N_USERS], summed[N_USERS:], out1, out2


# speedup vs baseline: 17.4323x; 1.2958x over previous
import functools

import jax
import jax.numpy as jnp
from jax import lax
from jax.experimental import pallas as pl
from jax.experimental.pallas import tpu as pltpu
from jax.experimental.pallas import tpu_sc as plsc

N_USERS = 5000
N_ITEMS = 5000
N = N_USERS + N_ITEMS
E = 320000
D = 128
NC = 2
NS = 16
NW = NC * NS
EPW = E // NW
CH = 80
SCH = 25
SCE = SCH * CH
NSUP = EPW // SCE
NSC = NW * NSUP
NPAD = 10240
RPT = NPAD // NW
SPT = NPAD // NS
APT = NPAD // NS
AZC = 80
CR = 80
G16 = 16

_mesh = plsc.VectorSubcoreMesh(core_axis_name="c", subcore_axis_name="s")
_params = pltpu.CompilerParams(needs_layout_passes=False)


def _rsqrt16(x):
    i = lax.bitcast_convert_type(x, jnp.int32)
    i = jnp.int32(0x5F3759DF) - (i >> 1)
    y = lax.bitcast_convert_type(i, jnp.float32)
    for _ in range(3):
        y = y * (1.5 - 0.5 * x * y * y)
    return y


@functools.partial(
    pl.kernel,
    out_type=jax.ShapeDtypeStruct((NC, NPAD), jnp.float32),
    mesh=_mesh,
    compiler_params=_params,
    scratch_types=[
        pltpu.VMEM((SCH, CH), jnp.int32),
        pltpu.VMEM((CH,), jnp.float32),
        pltpu.VMEM((SPT,), jnp.float32),
        pltpu.VMEM_SHARED((NPAD,), jnp.float32),
    ],
)
def _deg_kernel(h3_hbm, degp_hbm, h3s, ones_v, z_v, deg_sh):
    cid = lax.axis_index("c")
    sid = lax.axis_index("s")
    wid = sid * NC + cid

    def fill_ones(i, c):
        ones_v[pl.ds(i * G16, G16)] = jnp.full((G16,), 1.0, jnp.float32)
        return c

    lax.fori_loop(0, CH // G16, fill_ones, 0)

    def fill_zero(i, c):
        z_v[pl.ds(i * G16, G16)] = jnp.zeros((G16,), jnp.float32)
        return c

    lax.fori_loop(0, SPT // G16, fill_zero, 0)
    pltpu.sync_copy(z_v, deg_sh.at[pl.ds(sid * SPT, SPT)])
    plsc.subcore_barrier()

    for s in range(NSUP):
        pltpu.sync_copy(h3_hbm.at[wid * NSUP + s], h3s)

        def scat(j, c):
            pltpu.sync_copy(ones_v, deg_sh.at[h3s.at[j]], add=True)
            return c

        lax.fori_loop(0, SCH, scat, 0)
    plsc.subcore_barrier()
    pltpu.sync_copy(deg_sh.at[pl.ds(sid * SPT, SPT)], z_v)
    pltpu.sync_copy(z_v, degp_hbm.at[cid, pl.ds(sid * SPT, SPT)])


def _zero_acc(buf, acc_sh, sid):
    def zrow(r, c):
        for k in range(D // G16):
            buf[r, pl.ds(k * G16, G16)] = jnp.zeros((G16,), jnp.float32)
        return c

    lax.fori_loop(0, CH, zrow, 0)
    for i in range(APT // AZC):
        pltpu.sync_copy(buf, acc_sh.at[pl.ds(sid * APT + i * AZC, AZC)])


def _scale_rows(buf, g_v, j):
    base = j * CH

    def blk(q, c):
        gvec = g_v[pl.ds(base + q * G16, G16)]
        for r16 in range(G16):
            gb = jnp.full((G16,), gvec[r16], jnp.float32)
            row = q * G16 + r16
            for k in range(D // G16):
                buf[row, pl.ds(k * G16, G16)] = buf[row, pl.ds(k * G16, G16)] * gb
        return c

    lax.fori_loop(0, CH // G16, blk, 0)


def _spmm_super(x_hbm, h3s, t1s, g_v, bufs, acc_sh, gs, ss):
    def gather(j, buf, sem):
        off = pl.multiple_of(j * CH, 16)
        return pltpu.async_copy(x_hbm.at[t1s.at[pl.ds(off, CH)]], buf, sem)

    def gwait(buf, sem):
        pltpu.make_async_copy(x_hbm.at[t1s.at[pl.ds(0, CH)]], buf, sem).wait()

    def scat(j, buf, sem):
        return pltpu.async_copy(buf, acc_sh.at[h3s.at[j]], sem, add=True)

    def swait(buf, sem):
        pltpu.make_async_copy(buf, acc_sh.at[h3s.at[0]], sem).wait()

    def zrow(r, c):
        for k in range(D // G16):
            bufs[2][r, pl.ds(k * G16, G16)] = jnp.zeros((G16,), jnp.float32)
        return c

    lax.fori_loop(0, CH, zrow, 0)
    scat(0, bufs[2], ss[2])
    gather(0, bufs[0], gs[0])
    gather(1, bufs[1], gs[1])

    def step(j, b):
        y = (j + 2) % 3 if isinstance(j, int) else None
        gwait(bufs[b], gs[b])
        _scale_rows(bufs[b], g_v, j)
        yb = (b + 2) % 3
        swait(bufs[yb], ss[yb])
        return yb

    def triple(jj, c):
        j0 = jj * 3
        for b in range(3):
            j = j0 + b
            yb = step(j, b)
            gather(j + 2, bufs[yb], gs[yb])
            scat(j, bufs[b], ss[b])
        return c

    lax.fori_loop(0, (SCH - 4) // 3, triple, 0)
    for j in range(SCH - 4, SCH):
        b = j % 3
        yb = step(j, b)
        if j + 2 < SCH:
            gather(j + 2, bufs[yb], gs[yb])
        scat(j, bufs[b], ss[b])
    swait(bufs[(SCH - 1) % 3], ss[(SCH - 1) % 3])


def _write_partial(acc_sh, part_hbm, cid, sid):
    for i in range(APT // AZC):
        rows = pl.ds(sid * APT + i * AZC, AZC)
        pltpu.sync_copy(acc_sh.at[rows], part_hbm.at[cid, rows])


@functools.partial(
    pl.kernel,
    out_type=jax.ShapeDtypeStruct((NSC, SCE), jnp.float32),
    mesh=_mesh,
    compiler_params=_params,
    scratch_types=[
        pltpu.VMEM((SCE,), jnp.int32),
        pltpu.VMEM((SCE,), jnp.int32),
        pltpu.VMEM((SCE,), jnp.float32),
        pltpu.VMEM((NPAD,), jnp.float32),
        pltpu.VMEM((SPT,), jnp.float32),
    ],
)
def _g_kernel(hf_hbm, tf_hbm, degp_hbm, g_hbm, h1s, t1s, g_v, dis_v, dtmp):
    cid = lax.axis_index("c")
    sid = lax.axis_index("s")
    wid = sid * NC + cid

    pltpu.sync_copy(degp_hbm.at[0], dis_v)
    for p in range(NPAD // SPT):
        pltpu.sync_copy(degp_hbm.at[1, pl.ds(p * SPT, SPT)], dtmp)

        def disbody(i, c):
            sl = pl.ds(p * SPT + i * G16, G16)
            d = dis_v[sl] + dtmp[pl.ds(i * G16, G16)]
            r = _rsqrt16(jnp.maximum(d, 1.0))
            dis_v[sl] = jnp.where(d > 0.0, r, 0.0)
            return c

        lax.fori_loop(0, SPT // G16, disbody, 0)

    for s in range(NSUP):
        sc = wid * NSUP + s
        pltpu.sync_copy(hf_hbm.at[sc], h1s)
        pltpu.sync_copy(tf_hbm.at[sc], t1s)

        def gbody(i, c):
            sl = pl.ds(i * G16, G16)
            gh = plsc.load_gather(dis_v, [h1s[sl]])
            gt = plsc.load_gather(dis_v, [t1s[sl]])
            g_v[sl] = gh * gt
            return c

        lax.fori_loop(0, SCE // G16, gbody, 0)
        pltpu.sync_copy(g_v, g_hbm.at[sc])


@functools.partial(
    pl.kernel,
    out_type=jax.ShapeDtypeStruct((NC, NPAD, D), jnp.float32),
    mesh=_mesh,
    compiler_params=_params,
    scratch_types=[
        pltpu.VMEM((SCH, CH), jnp.int32),
        pltpu.VMEM((SCE,), jnp.int32),
        pltpu.VMEM((SCE,), jnp.float32),
        pltpu.VMEM((CH, D), jnp.float32),
        pltpu.VMEM((CH, D), jnp.float32),
        pltpu.VMEM((CH, D), jnp.float32),
        pltpu.VMEM_SHARED((NPAD, D), jnp.float32),
        pltpu.SemaphoreType.DMA,
        pltpu.SemaphoreType.DMA,
        pltpu.SemaphoreType.DMA,
        pltpu.SemaphoreType.DMA,
        pltpu.SemaphoreType.DMA,
        pltpu.SemaphoreType.DMA,
    ],
)
def _layer_kernel(x_hbm, h3_hbm, tf_hbm, g_hbm, part_hbm,
                  h3s, t1s, g_v, buf0, buf1, buf2, acc_sh,
                  gs0, gs1, gs2, ss0, ss1, ss2):
    cid = lax.axis_index("c")
    sid = lax.axis_index("s")
    wid = sid * NC + cid
    _zero_acc(buf0, acc_sh, sid)
    plsc.subcore_barrier()
    for s in range(NSUP):
        sc = wid * NSUP + s
        pltpu.sync_copy(h3_hbm.at[sc], h3s)
        pltpu.sync_copy(tf_hbm.at[sc], t1s)
        pltpu.sync_copy(g_hbm.at[sc], g_v)
        _spmm_super(x_hbm, h3s, t1s, g_v, (buf0, buf1, buf2), acc_sh,
                    (gs0, gs1, gs2), (ss0, ss1, ss2))
    plsc.subcore_barrier()
    _write_partial(acc_sh, part_hbm, cid, sid)


@functools.partial(
    pl.kernel,
    out_type=(
        jax.ShapeDtypeStruct((N, D), jnp.float32),
        jax.ShapeDtypeStruct((N, D), jnp.float32),
    ),
    mesh=_mesh,
    compiler_params=_params,
    scratch_types=[
        pltpu.VMEM((CR, D), jnp.float32),
        pltpu.VMEM((CR, D), jnp.float32),
        pltpu.VMEM((CR, D), jnp.float32),
    ],
)
def _combine1_kernel(part_hbm, x0_hbm, out1_hbm, emb1_hbm, pa, pb, px):
    cid = lax.axis_index("c")
    sid = lax.axis_index("s")
    wid = sid * NC + cid
    base = pl.multiple_of(jnp.minimum(wid * RPT, N - RPT), 16)
    for i in range(RPT // CR):
        b = base + i * CR
        pltpu.sync_copy(part_hbm.at[0, pl.ds(b, CR)], pa)
        pltpu.sync_copy(part_hbm.at[1, pl.ds(b, CR)], pb)
        pltpu.sync_copy(x0_hbm.at[pl.ds(b, CR)], px)

        def rb(r, c):
            for k in range(D // G16):
                sl = pl.ds(k * G16, G16)
                sm = pa[r, sl] + pb[r, sl]
                pa[r, sl] = sm
                px[r, sl] = sm + px[r, sl]
            return c

        lax.fori_loop(0, CR, rb, 0)
        pltpu.sync_copy(pa, out1_hbm.at[pl.ds(b, CR)])
        pltpu.sync_copy(px, emb1_hbm.at[pl.ds(b, CR)])


@functools.partial(
    pl.kernel,
    out_type=(
        jax.ShapeDtypeStruct((N, D), jnp.float32),
        jax.ShapeDtypeStruct((N, D), jnp.float32),
    ),
    mesh=_mesh,
    compiler_params=_params,
    scratch_types=[
        pltpu.VMEM((CR, D), jnp.float32),
        pltpu.VMEM((CR, D), jnp.float32),
        pltpu.VMEM((CR, D), jnp.float32),
        pltpu.VMEM((CR, D), jnp.float32),
    ],
)
def _combine2_kernel(part_hbm, x0_hbm, emb1_hbm, out2_hbm, summed_hbm,
                     pa, pb, px, pe):
    cid = lax.axis_index("c")
    sid = lax.axis_index("s")
    wid = sid * NC + cid
    base = pl.multiple_of(jnp.minimum(wid * RPT, N - RPT), 16)
    for i in range(RPT // CR):
        b = base + i * CR
        pltpu.sync_copy(part_hbm.at[0, pl.ds(b, CR)], pa)
        pltpu.sync_copy(part_hbm.at[1, pl.ds(b, CR)], pb)
        pltpu.sync_copy(x0_hbm.at[pl.ds(b, CR)], px)
        pltpu.sync_copy(emb1_hbm.at[pl.ds(b, CR)], pe)

        def rb(r, c):
            for k in range(D // G16):
                sl = pl.ds(k * G16, G16)
                o2 = pa[r, sl] + pb[r, sl]
                pa[r, sl] = o2
                pe[r, sl] = px[r, sl] + 2.0 * pe[r, sl] + o2
            return c

        lax.fori_loop(0, CR, rb, 0)
        pltpu.sync_copy(pa, out2_hbm.at[pl.ds(b, CR)])
        pltpu.sync_copy(pe, summed_hbm.at[pl.ds(b, CR)])


def kernel(user_emb, item_emb, h_list, t_list):
    x0 = jnp.concatenate([user_emb, item_emb], axis=0)
    h3 = h_list.reshape(NSC, SCH, CH)
    hf = h_list.reshape(NSC, SCE)
    tf = t_list.reshape(NSC, SCE)
    degp = _deg_kernel(h3)
    g = _g_kernel(hf, tf, degp)
    part1 = _layer_kernel(x0, h3, tf, g)
    out1, emb1 = _combine1_kernel(part1, x0)
    part2 = _layer_kernel(emb1, h3, tf, g)
    out2, summed = _combine2_kernel(part2, x0, emb1)
    return summed[:N_USERS], summed[N_USERS:], out1, out2


# speedup vs baseline: 17.7349x; 1.0174x over previous
import functools

import jax
import jax.numpy as jnp
from jax import lax
from jax.experimental import pallas as pl
from jax.experimental.pallas import tpu as pltpu
from jax.experimental.pallas import tpu_sc as plsc

N_USERS = 5000
N_ITEMS = 5000
N = N_USERS + N_ITEMS
E = 320000
D = 128
NC = 2
NS = 16
NW = NC * NS
EPW = E // NW
CH = 80
SCH = 25
SCE = SCH * CH
NSUP = EPW // SCE
NSC = NW * NSUP
NPAD = 10240
RPT = NPAD // NW
SPT = NPAD // NS
APT = NPAD // NS
AZC = 80
CR = 80
G16 = 16

_mesh = plsc.VectorSubcoreMesh(core_axis_name="c", subcore_axis_name="s")
_params = pltpu.CompilerParams(needs_layout_passes=False)


def _rsqrt16(x):
    i = lax.bitcast_convert_type(x, jnp.int32)
    i = jnp.int32(0x5F3759DF) - (i >> 1)
    y = lax.bitcast_convert_type(i, jnp.float32)
    for _ in range(3):
        y = y * (1.5 - 0.5 * x * y * y)
    return y


@functools.partial(
    pl.kernel,
    out_type=jax.ShapeDtypeStruct((NC, NPAD), jnp.float32),
    mesh=_mesh,
    compiler_params=_params,
    scratch_types=[
        pltpu.VMEM((SCH, CH), jnp.int32),
        pltpu.VMEM((CH,), jnp.float32),
        pltpu.VMEM((SPT,), jnp.float32),
        pltpu.VMEM_SHARED((NPAD,), jnp.float32),
    ],
)
def _deg_kernel(h3_hbm, degp_hbm, h3s, ones_v, z_v, deg_sh):
    cid = lax.axis_index("c")
    sid = lax.axis_index("s")
    wid = sid * NC + cid

    def fill_ones(i, c):
        ones_v[pl.ds(i * G16, G16)] = jnp.full((G16,), 1.0, jnp.float32)
        return c

    lax.fori_loop(0, CH // G16, fill_ones, 0)

    def fill_zero(i, c):
        z_v[pl.ds(i * G16, G16)] = jnp.zeros((G16,), jnp.float32)
        return c

    lax.fori_loop(0, SPT // G16, fill_zero, 0)
    pltpu.sync_copy(z_v, deg_sh.at[pl.ds(sid * SPT, SPT)])
    plsc.subcore_barrier()

    for s in range(NSUP):
        pltpu.sync_copy(h3_hbm.at[wid * NSUP + s], h3s)

        def scat(j, c):
            pltpu.sync_copy(ones_v, deg_sh.at[h3s.at[j]], add=True)
            return c

        lax.fori_loop(0, SCH, scat, 0)
    plsc.subcore_barrier()
    pltpu.sync_copy(deg_sh.at[pl.ds(sid * SPT, SPT)], z_v)
    pltpu.sync_copy(z_v, degp_hbm.at[cid, pl.ds(sid * SPT, SPT)])


def _zero_acc(buf, acc_sh, sid):
    def zrow(r, c):
        for k in range(D // G16):
            buf[r, pl.ds(k * G16, G16)] = jnp.zeros((G16,), jnp.float32)
        return c

    lax.fori_loop(0, CH, zrow, 0)
    for i in range(APT // AZC):
        pltpu.sync_copy(buf, acc_sh.at[pl.ds(sid * APT + i * AZC, AZC)])


def _scale_rows(buf, g_v, j):
    base = j * CH

    def blk(q, c):
        gvec = g_v[pl.ds(base + q * G16, G16)]
        for r16 in range(G16):
            gb = jnp.full((G16,), gvec[r16], jnp.float32)
            row = q * G16 + r16
            for k in range(D // G16):
                buf[row, pl.ds(k * G16, G16)] = buf[row, pl.ds(k * G16, G16)] * gb
        return c

    lax.fori_loop(0, CH // G16, blk, 0)


def _spmm_super(x_hbm, h3s, t1s, g_v, bufs, acc_sh, gs, ss):
    NB = 4

    def gather(j, buf, sem):
        off = pl.multiple_of(j * CH, 16)
        return pltpu.async_copy(x_hbm.at[t1s.at[pl.ds(off, CH)]], buf, sem)

    def gwait(buf, sem):
        pltpu.make_async_copy(x_hbm.at[t1s.at[pl.ds(0, CH)]], buf, sem).wait()

    def scat(j, buf, sem):
        return pltpu.async_copy(buf, acc_sh.at[h3s.at[j]], sem, add=True)

    def swait(buf, sem):
        pltpu.make_async_copy(buf, acc_sh.at[h3s.at[0]], sem).wait()

    def zrow(r, c):
        for k in range(D // G16):
            bufs[NB - 1][r, pl.ds(k * G16, G16)] = jnp.zeros((G16,), jnp.float32)
        return c

    lax.fori_loop(0, CH, zrow, 0)
    scat(0, bufs[NB - 1], ss[NB - 1])
    for b in range(NB - 1):
        gather(b, bufs[b], gs[b])

    def step(j, b):
        gwait(bufs[b], gs[b])
        _scale_rows(bufs[b], g_v, j)
        yb = (b + NB - 1) % NB
        swait(bufs[yb], ss[yb])
        return yb

    def group(jj, c):
        j0 = jj * NB
        for b in range(NB):
            j = j0 + b
            yb = step(j, b)
            gather(j + NB - 1, bufs[yb], gs[yb])
            scat(j, bufs[b], ss[b])
        return c

    npeel = NB + (SCH % NB)
    lax.fori_loop(0, (SCH - npeel) // NB, group, 0)
    for j in range(SCH - npeel, SCH):
        b = j % NB
        yb = step(j, b)
        if j + NB - 1 < SCH:
            gather(j + NB - 1, bufs[yb], gs[yb])
        scat(j, bufs[b], ss[b])
    swait(bufs[(SCH - 1) % NB], ss[(SCH - 1) % NB])


def _write_partial(acc_sh, part_hbm, cid, sid):
    for i in range(APT // AZC):
        rows = pl.ds(sid * APT + i * AZC, AZC)
        pltpu.sync_copy(acc_sh.at[rows], part_hbm.at[cid, rows])


@functools.partial(
    pl.kernel,
    out_type=jax.ShapeDtypeStruct((NSC, SCE), jnp.float32),
    mesh=_mesh,
    compiler_params=_params,
    scratch_types=[
        pltpu.VMEM((SCE,), jnp.int32),
        pltpu.VMEM((SCE,), jnp.int32),
        pltpu.VMEM((SCE,), jnp.float32),
        pltpu.VMEM((NPAD,), jnp.float32),
        pltpu.VMEM((SPT,), jnp.float32),
    ],
)
def _g_kernel(hf_hbm, tf_hbm, degp_hbm, g_hbm, h1s, t1s, g_v, dis_v, dtmp):
    cid = lax.axis_index("c")
    sid = lax.axis_index("s")
    wid = sid * NC + cid

    pltpu.sync_copy(degp_hbm.at[0], dis_v)
    for p in range(NPAD // SPT):
        pltpu.sync_copy(degp_hbm.at[1, pl.ds(p * SPT, SPT)], dtmp)

        def disbody(i, c):
            sl = pl.ds(p * SPT + i * G16, G16)
            d = dis_v[sl] + dtmp[pl.ds(i * G16, G16)]
            r = _rsqrt16(jnp.maximum(d, 1.0))
            dis_v[sl] = jnp.where(d > 0.0, r, 0.0)
            return c

        lax.fori_loop(0, SPT // G16, disbody, 0)

    for s in range(NSUP):
        sc = wid * NSUP + s
        pltpu.sync_copy(hf_hbm.at[sc], h1s)
        pltpu.sync_copy(tf_hbm.at[sc], t1s)

        def gbody(i, c):
            sl = pl.ds(i * G16, G16)
            gh = plsc.load_gather(dis_v, [h1s[sl]])
            gt = plsc.load_gather(dis_v, [t1s[sl]])
            g_v[sl] = gh * gt
            return c

        lax.fori_loop(0, SCE // G16, gbody, 0)
        pltpu.sync_copy(g_v, g_hbm.at[sc])


@functools.partial(
    pl.kernel,
    out_type=jax.ShapeDtypeStruct((NC, NPAD, D), jnp.float32),
    mesh=_mesh,
    compiler_params=_params,
    scratch_types=[
        pltpu.VMEM((SCH, CH), jnp.int32),
        pltpu.VMEM((SCE,), jnp.int32),
        pltpu.VMEM((SCE,), jnp.float32),
        pltpu.VMEM((CH, D), jnp.float32),
        pltpu.VMEM((CH, D), jnp.float32),
        pltpu.VMEM((CH, D), jnp.float32),
        pltpu.VMEM((CH, D), jnp.float32),
        pltpu.VMEM_SHARED((NPAD, D), jnp.float32),
        pltpu.SemaphoreType.DMA,
        pltpu.SemaphoreType.DMA,
        pltpu.SemaphoreType.DMA,
        pltpu.SemaphoreType.DMA,
        pltpu.SemaphoreType.DMA,
        pltpu.SemaphoreType.DMA,
        pltpu.SemaphoreType.DMA,
        pltpu.SemaphoreType.DMA,
    ],
)
def _layer_kernel(x_hbm, h3_hbm, tf_hbm, g_hbm, part_hbm,
                  h3s, t1s, g_v, buf0, buf1, buf2, buf3, acc_sh,
                  gs0, gs1, gs2, gs3, ss0, ss1, ss2, ss3):
    cid = lax.axis_index("c")
    sid = lax.axis_index("s")
    wid = sid * NC + cid
    _zero_acc(buf0, acc_sh, sid)
    plsc.subcore_barrier()
    for s in range(NSUP):
        sc = wid * NSUP + s
        pltpu.sync_copy(h3_hbm.at[sc], h3s)
        pltpu.sync_copy(tf_hbm.at[sc], t1s)
        pltpu.sync_copy(g_hbm.at[sc], g_v)
        _spmm_super(x_hbm, h3s, t1s, g_v, (buf0, buf1, buf2, buf3), acc_sh,
                    (gs0, gs1, gs2, gs3), (ss0, ss1, ss2, ss3))
    plsc.subcore_barrier()
    _write_partial(acc_sh, part_hbm, cid, sid)


@functools.partial(
    pl.kernel,
    out_type=(
        jax.ShapeDtypeStruct((N, D), jnp.float32),
        jax.ShapeDtypeStruct((N, D), jnp.float32),
    ),
    mesh=_mesh,
    compiler_params=_params,
    scratch_types=[
        pltpu.VMEM((CR, D), jnp.float32),
        pltpu.VMEM((CR, D), jnp.float32),
        pltpu.VMEM((CR, D), jnp.float32),
    ],
)
def _combine1_kernel(part_hbm, x0_hbm, out1_hbm, emb1_hbm, pa, pb, px):
    cid = lax.axis_index("c")
    sid = lax.axis_index("s")
    wid = sid * NC + cid
    base = pl.multiple_of(jnp.minimum(wid * RPT, N - RPT), 16)
    for i in range(RPT // CR):
        b = base + i * CR
        pltpu.sync_copy(part_hbm.at[0, pl.ds(b, CR)], pa)
        pltpu.sync_copy(part_hbm.at[1, pl.ds(b, CR)], pb)
        pltpu.sync_copy(x0_hbm.at[pl.ds(b, CR)], px)

        def rb(r, c):
            for k in range(D // G16):
                sl = pl.ds(k * G16, G16)
                sm = pa[r, sl] + pb[r, sl]
                pa[r, sl] = sm
                px[r, sl] = sm + px[r, sl]
            return c

        lax.fori_loop(0, CR, rb, 0)
        pltpu.sync_copy(pa, out1_hbm.at[pl.ds(b, CR)])
        pltpu.sync_copy(px, emb1_hbm.at[pl.ds(b, CR)])


@functools.partial(
    pl.kernel,
    out_type=(
        jax.ShapeDtypeStruct((N, D), jnp.float32),
        jax.ShapeDtypeStruct((N, D), jnp.float32),
    ),
    mesh=_mesh,
    compiler_params=_params,
    scratch_types=[
        pltpu.VMEM((CR, D), jnp.float32),
        pltpu.VMEM((CR, D), jnp.float32),
        pltpu.VMEM((CR, D), jnp.float32),
        pltpu.VMEM((CR, D), jnp.float32),
    ],
)
def _combine2_kernel(part_hbm, x0_hbm, emb1_hbm, out2_hbm, summed_hbm,
                     pa, pb, px, pe):
    cid = lax.axis_index("c")
    sid = lax.axis_index("s")
    wid = sid * NC + cid
    base = pl.multiple_of(jnp.minimum(wid * RPT, N - RPT), 16)
    for i in range(RPT // CR):
        b = base + i * CR
        pltpu.sync_copy(part_hbm.at[0, pl.ds(b, CR)], pa)
        pltpu.sync_copy(part_hbm.at[1, pl.ds(b, CR)], pb)
        pltpu.sync_copy(x0_hbm.at[pl.ds(b, CR)], px)
        pltpu.sync_copy(emb1_hbm.at[pl.ds(b, CR)], pe)

        def rb(r, c):
            for k in range(D // G16):
                sl = pl.ds(k * G16, G16)
                o2 = pa[r, sl] + pb[r, sl]
                pa[r, sl] = o2
                pe[r, sl] = px[r, sl] + 2.0 * pe[r, sl] + o2
            return c

        lax.fori_loop(0, CR, rb, 0)
        pltpu.sync_copy(pa, out2_hbm.at[pl.ds(b, CR)])
        pltpu.sync_copy(pe, summed_hbm.at[pl.ds(b, CR)])


def kernel(user_emb, item_emb, h_list, t_list):
    x0 = jnp.concatenate([user_emb, item_emb], axis=0)
    h3 = h_list.reshape(NSC, SCH, CH)
    hf = h_list.reshape(NSC, SCE)
    tf = t_list.reshape(NSC, SCE)
    degp = _deg_kernel(h3)
    g = _g_kernel(hf, tf, degp)
    part1 = _layer_kernel(x0, h3, tf, g)
    out1, emb1 = _combine1_kernel(part1, x0)
    part2 = _layer_kernel(emb1, h3, tf, g)
    out2, summed = _combine2_kernel(part2, x0, emb1)
    return summed[:N_USERS], summed[N_USERS:], out1, out2


# speedup vs baseline: 17.9327x; 1.0112x over previous
import functools

import jax
import jax.numpy as jnp
from jax import lax
from jax.experimental import pallas as pl
from jax.experimental.pallas import tpu as pltpu
from jax.experimental.pallas import tpu_sc as plsc

N_USERS = 5000
N_ITEMS = 5000
N = N_USERS + N_ITEMS
E = 320000
D = 128
NC = 2
NS = 16
NW = NC * NS
EPW = E // NW
CH = 80
SCH = 25
SCE = SCH * CH
NSUP = EPW // SCE
NSC = NW * NSUP
NPAD = 10240
RPT = NPAD // NW
SPT = NPAD // NS
APT = NPAD // NS
AZC = 80
CR = 80
G16 = 16

_mesh = plsc.VectorSubcoreMesh(core_axis_name="c", subcore_axis_name="s")
_params = pltpu.CompilerParams(needs_layout_passes=False)


def _rsqrt16(x):
    i = lax.bitcast_convert_type(x, jnp.int32)
    i = jnp.int32(0x5F3759DF) - (i >> 1)
    y = lax.bitcast_convert_type(i, jnp.float32)
    for _ in range(3):
        y = y * (1.5 - 0.5 * x * y * y)
    return y


@functools.partial(
    pl.kernel,
    out_type=jax.ShapeDtypeStruct((NC, NPAD), jnp.float32),
    mesh=_mesh,
    compiler_params=_params,
    scratch_types=[
        pltpu.VMEM((SCH, CH), jnp.int32),
        pltpu.VMEM((CH,), jnp.float32),
        pltpu.VMEM((SPT,), jnp.float32),
        pltpu.VMEM_SHARED((NPAD,), jnp.float32),
    ],
)
def _deg_kernel(h3_hbm, degp_hbm, h3s, ones_v, z_v, deg_sh):
    cid = lax.axis_index("c")
    sid = lax.axis_index("s")
    wid = sid * NC + cid

    def fill_ones(i, c):
        ones_v[pl.ds(i * G16, G16)] = jnp.full((G16,), 1.0, jnp.float32)
        return c

    lax.fori_loop(0, CH // G16, fill_ones, 0)

    def fill_zero(i, c):
        z_v[pl.ds(i * G16, G16)] = jnp.zeros((G16,), jnp.float32)
        return c

    lax.fori_loop(0, SPT // G16, fill_zero, 0)
    pltpu.sync_copy(z_v, deg_sh.at[pl.ds(sid * SPT, SPT)])
    plsc.subcore_barrier()

    for s in range(NSUP):
        pltpu.sync_copy(h3_hbm.at[wid * NSUP + s], h3s)

        def scat(j, c):
            pltpu.sync_copy(ones_v, deg_sh.at[h3s.at[j]], add=True)
            return c

        lax.fori_loop(0, SCH, scat, 0)
    plsc.subcore_barrier()
    pltpu.sync_copy(deg_sh.at[pl.ds(sid * SPT, SPT)], z_v)
    pltpu.sync_copy(z_v, degp_hbm.at[cid, pl.ds(sid * SPT, SPT)])


def _zero_acc(buf, acc_sh, sid):
    def zrow(r, c):
        for k in range(D // G16):
            buf[r, pl.ds(k * G16, G16)] = jnp.zeros((G16,), jnp.float32)
        return c

    lax.fori_loop(0, CH, zrow, 0)
    for i in range(APT // AZC):
        pltpu.sync_copy(buf, acc_sh.at[pl.ds(sid * APT + i * AZC, AZC)])


def _scale_rows(buf, g_v, j):
    base = j * CH

    def blk(q, c):
        gvec = g_v[pl.ds(base + q * G16, G16)]
        for r16 in range(G16):
            gb = jnp.full((G16,), gvec[r16], jnp.float32)
            row = q * G16 + r16
            for k in range(D // G16):
                buf[row, pl.ds(k * G16, G16)] = buf[row, pl.ds(k * G16, G16)] * gb
        return c

    lax.fori_loop(0, CH // G16, blk, 0)


def _spmm_super(x_hbm, h3s, t1s, g_v, bufs, acc_sh, gs, ss):
    def gather(j, buf, sem):
        off = pl.multiple_of(j * CH, 16)
        return pltpu.async_copy(x_hbm.at[t1s.at[pl.ds(off, CH)]], buf, sem)

    def gwait(buf, sem):
        pltpu.make_async_copy(x_hbm.at[t1s.at[pl.ds(0, CH)]], buf, sem).wait()

    def scat(j, buf, sem):
        return pltpu.async_copy(buf, acc_sh.at[h3s.at[j]], sem, add=True)

    def swait(buf, sem):
        pltpu.make_async_copy(buf, acc_sh.at[h3s.at[0]], sem).wait()

    def step(j, b, with_swait, with_gather):
        gwait(bufs[b], gs[b])
        yb = (b + 2) % 4
        if with_swait:
            swait(bufs[yb], ss[yb])
        if with_gather:
            gather(j + 2, bufs[yb], gs[yb])
        _scale_rows(bufs[b], g_v, j)
        scat(j, bufs[b], ss[b])

    gather(0, bufs[0], gs[0])
    gather(1, bufs[1], gs[1])
    step(0, 0, False, True)
    step(1, 1, False, True)

    def quad(jj, c):
        j0 = jj * 4 + 2
        for i, b in enumerate((2, 3, 0, 1)):
            step(j0 + i, b, True, True)
        return c

    lax.fori_loop(0, (SCH - 5) // 4, quad, 0)
    step(SCH - 3, 2, True, True)
    step(SCH - 2, 3, True, False)
    step(SCH - 1, 0, True, False)
    swait(bufs[3], ss[3])
    swait(bufs[0], ss[0])


def _write_partial(acc_sh, part_hbm, cid, sid):
    for i in range(APT // AZC):
        rows = pl.ds(sid * APT + i * AZC, AZC)
        pltpu.sync_copy(acc_sh.at[rows], part_hbm.at[cid, rows])


@functools.partial(
    pl.kernel,
    out_type=jax.ShapeDtypeStruct((NSC, SCE), jnp.float32),
    mesh=_mesh,
    compiler_params=_params,
    scratch_types=[
        pltpu.VMEM((SCE,), jnp.int32),
        pltpu.VMEM((SCE,), jnp.int32),
        pltpu.VMEM((SCE,), jnp.float32),
        pltpu.VMEM((NPAD,), jnp.float32),
        pltpu.VMEM((SPT,), jnp.float32),
    ],
)
def _g_kernel(hf_hbm, tf_hbm, degp_hbm, g_hbm, h1s, t1s, g_v, dis_v, dtmp):
    cid = lax.axis_index("c")
    sid = lax.axis_index("s")
    wid = sid * NC + cid

    pltpu.sync_copy(degp_hbm.at[0], dis_v)
    for p in range(NPAD // SPT):
        pltpu.sync_copy(degp_hbm.at[1, pl.ds(p * SPT, SPT)], dtmp)

        def disbody(i, c):
            sl = pl.ds(p * SPT + i * G16, G16)
            d = dis_v[sl] + dtmp[pl.ds(i * G16, G16)]
            r = _rsqrt16(jnp.maximum(d, 1.0))
            dis_v[sl] = jnp.where(d > 0.0, r, 0.0)
            return c

        lax.fori_loop(0, SPT // G16, disbody, 0)

    for s in range(NSUP):
        sc = wid * NSUP + s
        pltpu.sync_copy(hf_hbm.at[sc], h1s)
        pltpu.sync_copy(tf_hbm.at[sc], t1s)

        def gbody(i, c):
            sl = pl.ds(i * G16, G16)
            gh = plsc.load_gather(dis_v, [h1s[sl]])
            gt = plsc.load_gather(dis_v, [t1s[sl]])
            g_v[sl] = gh * gt
            return c

        lax.fori_loop(0, SCE // G16, gbody, 0)
        pltpu.sync_copy(g_v, g_hbm.at[sc])


@functools.partial(
    pl.kernel,
    out_type=jax.ShapeDtypeStruct((NC, NPAD, D), jnp.float32),
    mesh=_mesh,
    compiler_params=_params,
    scratch_types=[
        pltpu.VMEM((SCH, CH), jnp.int32),
        pltpu.VMEM((SCE,), jnp.int32),
        pltpu.VMEM((SCE,), jnp.float32),
        pltpu.VMEM((CH, D), jnp.float32),
        pltpu.VMEM((CH, D), jnp.float32),
        pltpu.VMEM((CH, D), jnp.float32),
        pltpu.VMEM((CH, D), jnp.float32),
        pltpu.VMEM_SHARED((NPAD, D), jnp.float32),
        pltpu.SemaphoreType.DMA,
        pltpu.SemaphoreType.DMA,
        pltpu.SemaphoreType.DMA,
        pltpu.SemaphoreType.DMA,
        pltpu.SemaphoreType.DMA,
        pltpu.SemaphoreType.DMA,
        pltpu.SemaphoreType.DMA,
        pltpu.SemaphoreType.DMA,
    ],
)
def _layer_kernel(x_hbm, h3_hbm, tf_hbm, g_hbm, part_hbm,
                  h3s, t1s, g_v, buf0, buf1, buf2, buf3, acc_sh,
                  gs0, gs1, gs2, gs3, ss0, ss1, ss2, ss3):
    cid = lax.axis_index("c")
    sid = lax.axis_index("s")
    wid = sid * NC + cid
    _zero_acc(buf0, acc_sh, sid)
    plsc.subcore_barrier()
    for s in range(NSUP):
        sc = wid * NSUP + s
        pltpu.sync_copy(h3_hbm.at[sc], h3s)
        pltpu.sync_copy(tf_hbm.at[sc], t1s)
        pltpu.sync_copy(g_hbm.at[sc], g_v)
        _spmm_super(x_hbm, h3s, t1s, g_v, (buf0, buf1, buf2, buf3), acc_sh,
                    (gs0, gs1, gs2, gs3), (ss0, ss1, ss2, ss3))
    plsc.subcore_barrier()
    _write_partial(acc_sh, part_hbm, cid, sid)


@functools.partial(
    pl.kernel,
    out_type=(
        jax.ShapeDtypeStruct((N, D), jnp.float32),
        jax.ShapeDtypeStruct((N, D), jnp.float32),
    ),
    mesh=_mesh,
    compiler_params=_params,
    scratch_types=[
        pltpu.VMEM((CR, D), jnp.float32),
        pltpu.VMEM((CR, D), jnp.float32),
        pltpu.VMEM((CR, D), jnp.float32),
    ],
)
def _combine1_kernel(part_hbm, x0_hbm, out1_hbm, emb1_hbm, pa, pb, px):
    cid = lax.axis_index("c")
    sid = lax.axis_index("s")
    wid = sid * NC + cid
    base = pl.multiple_of(jnp.minimum(wid * RPT, N - RPT), 16)
    for i in range(RPT // CR):
        b = base + i * CR
        pltpu.sync_copy(part_hbm.at[0, pl.ds(b, CR)], pa)
        pltpu.sync_copy(part_hbm.at[1, pl.ds(b, CR)], pb)
        pltpu.sync_copy(x0_hbm.at[pl.ds(b, CR)], px)

        def rb(r, c):
            for k in range(D // G16):
                sl = pl.ds(k * G16, G16)
                sm = pa[r, sl] + pb[r, sl]
                pa[r, sl] = sm
                px[r, sl] = sm + px[r, sl]
            return c

        lax.fori_loop(0, CR, rb, 0)
        pltpu.sync_copy(pa, out1_hbm.at[pl.ds(b, CR)])
        pltpu.sync_copy(px, emb1_hbm.at[pl.ds(b, CR)])


@functools.partial(
    pl.kernel,
    out_type=(
        jax.ShapeDtypeStruct((N, D), jnp.float32),
        jax.ShapeDtypeStruct((N, D), jnp.float32),
    ),
    mesh=_mesh,
    compiler_params=_params,
    scratch_types=[
        pltpu.VMEM((CR, D), jnp.float32),
        pltpu.VMEM((CR, D), jnp.float32),
        pltpu.VMEM((CR, D), jnp.float32),
        pltpu.VMEM((CR, D), jnp.float32),
    ],
)
def _combine2_kernel(part_hbm, x0_hbm, emb1_hbm, out2_hbm, summed_hbm,
                     pa, pb, px, pe):
    cid = lax.axis_index("c")
    sid = lax.axis_index("s")
    wid = sid * NC + cid
    base = pl.multiple_of(jnp.minimum(wid * RPT, N - RPT), 16)
    for i in range(RPT // CR):
        b = base + i * CR
        pltpu.sync_copy(part_hbm.at[0, pl.ds(b, CR)], pa)
        pltpu.sync_copy(part_hbm.at[1, pl.ds(b, CR)], pb)
        pltpu.sync_copy(x0_hbm.at[pl.ds(b, CR)], px)
        pltpu.sync_copy(emb1_hbm.at[pl.ds(b, CR)], pe)

        def rb(r, c):
            for k in range(D // G16):
                sl = pl.ds(k * G16, G16)
                o2 = pa[r, sl] + pb[r, sl]
                pa[r, sl] = o2
                pe[r, sl] = px[r, sl] + 2.0 * pe[r, sl] + o2
            return c

        lax.fori_loop(0, CR, rb, 0)
        pltpu.sync_copy(pa, out2_hbm.at[pl.ds(b, CR)])
        pltpu.sync_copy(pe, summed_hbm.at[pl.ds(b, CR)])


def kernel(user_emb, item_emb, h_list, t_list):
    x0 = jnp.concatenate([user_emb, item_emb], axis=0)
    h3 = h_list.reshape(NSC, SCH, CH)
    hf = h_list.reshape(NSC, SCE)
    tf = t_list.reshape(NSC, SCE)
    degp = _deg_kernel(h3)
    g = _g_kernel(hf, tf, degp)
    part1 = _layer_kernel(x0, h3, tf, g)
    out1, emb1 = _combine1_kernel(part1, x0)
    part2 = _layer_kernel(emb1, h3, tf, g)
    out2, summed = _combine2_kernel(part2, x0, emb1)
    return summed[:N_USERS], summed[N_USERS:], out1, out2


# speedup vs baseline: 19.0419x; 1.0619x over previous
import functools

import jax
import jax.numpy as jnp
from jax import lax
from jax.experimental import pallas as pl
from jax.experimental.pallas import tpu as pltpu
from jax.experimental.pallas import tpu_sc as plsc

N_USERS = 5000
N_ITEMS = 5000
N = N_USERS + N_ITEMS
E = 320000
D = 128
NC = 2
NS = 16
NW = NC * NS
EPW = E // NW
CH = 80
SCH = 25
SCE = SCH * CH
NSUP = EPW // SCE
NSC = NW * NSUP
NPAD = 10240
RPT = NPAD // NW
SPT = NPAD // NS
APT = NPAD // NS
AZC = 80
CR = 80
G16 = 16

_mesh = plsc.VectorSubcoreMesh(core_axis_name="c", subcore_axis_name="s")
_params = pltpu.CompilerParams(needs_layout_passes=False)


def _rsqrt16(x):
    i = lax.bitcast_convert_type(x, jnp.int32)
    i = jnp.int32(0x5F3759DF) - (i >> 1)
    y = lax.bitcast_convert_type(i, jnp.float32)
    for _ in range(3):
        y = y * (1.5 - 0.5 * x * y * y)
    return y


@functools.partial(
    pl.kernel,
    out_type=jax.ShapeDtypeStruct((NC, NPAD), jnp.float32),
    mesh=_mesh,
    compiler_params=_params,
    scratch_types=[
        pltpu.VMEM((SCH, CH), jnp.int32),
        pltpu.VMEM((CH,), jnp.float32),
        pltpu.VMEM((SPT,), jnp.float32),
        pltpu.VMEM_SHARED((NPAD,), jnp.float32),
    ],
)
def _deg_kernel(h3_hbm, degp_hbm, h3s, ones_v, z_v, deg_sh):
    cid = lax.axis_index("c")
    sid = lax.axis_index("s")
    wid = sid * NC + cid

    def fill_ones(i, c):
        ones_v[pl.ds(i * G16, G16)] = jnp.full((G16,), 1.0, jnp.float32)
        return c

    lax.fori_loop(0, CH // G16, fill_ones, 0)

    def fill_zero(i, c):
        z_v[pl.ds(i * G16, G16)] = jnp.zeros((G16,), jnp.float32)
        return c

    lax.fori_loop(0, SPT // G16, fill_zero, 0)
    pltpu.sync_copy(z_v, deg_sh.at[pl.ds(sid * SPT, SPT)])
    plsc.subcore_barrier()

    for s in range(NSUP):
        pltpu.sync_copy(h3_hbm.at[wid * NSUP + s], h3s)

        def scat(j, c):
            pltpu.sync_copy(ones_v, deg_sh.at[h3s.at[j]], add=True)
            return c

        lax.fori_loop(0, SCH, scat, 0)
    plsc.subcore_barrier()
    pltpu.sync_copy(deg_sh.at[pl.ds(sid * SPT, SPT)], z_v)
    pltpu.sync_copy(z_v, degp_hbm.at[cid, pl.ds(sid * SPT, SPT)])


def _zero_acc(buf, acc_sh, sid):
    def zrow(r, c):
        for k in range(D // G16):
            buf[r, pl.ds(k * G16, G16)] = jnp.zeros((G16,), jnp.float32)
        return c

    lax.fori_loop(0, CH, zrow, 0)
    for i in range(APT // AZC):
        pltpu.sync_copy(buf, acc_sh.at[pl.ds(sid * APT + i * AZC, AZC)])


def _scale_rows(buf, g_v, j):
    base = j * CH

    def blk(q, c):
        gvec = g_v[pl.ds(base + q * G16, G16)]
        for r16 in range(G16):
            gb = jnp.full((G16,), gvec[r16], jnp.float32)
            row = q * G16 + r16
            for k in range(D // G16):
                buf[row, pl.ds(k * G16, G16)] = buf[row, pl.ds(k * G16, G16)] * gb
        return c

    lax.fori_loop(0, CH // G16, blk, 0)


def _spmm_super(x_hbm, h3s, t1s, g_v, bufs, acc_sh, gs, ss):
    def gather(j, buf, sem):
        off = pl.multiple_of(j * CH, 16)
        return pltpu.async_copy(x_hbm.at[t1s.at[pl.ds(off, CH)]], buf, sem)

    def gwait(buf, sem):
        pltpu.make_async_copy(x_hbm.at[t1s.at[pl.ds(0, CH)]], buf, sem).wait()

    def scat(j, buf, sem):
        return pltpu.async_copy(buf, acc_sh.at[h3s.at[j]], sem, add=True)

    def swait(buf, sem):
        pltpu.make_async_copy(buf, acc_sh.at[h3s.at[0]], sem).wait()

    def step(j, b, with_swait, with_gather):
        gwait(bufs[b], gs[b])
        yb = (b + 2) % 4
        if with_swait:
            swait(bufs[yb], ss[yb])
        if with_gather:
            gather(j + 2, bufs[yb], gs[yb])
        _scale_rows(bufs[b], g_v, j)
        scat(j, bufs[b], ss[b])

    gather(0, bufs[0], gs[0])
    gather(1, bufs[1], gs[1])
    step(0, 0, False, True)
    step(1, 1, False, True)

    def quad(jj, c):
        j0 = jj * 4 + 2
        for i, b in enumerate((2, 3, 0, 1)):
            step(j0 + i, b, True, True)
        return c

    lax.fori_loop(0, (SCH - 5) // 4, quad, 0)
    step(SCH - 3, 2, True, True)
    step(SCH - 2, 3, True, False)
    step(SCH - 1, 0, True, False)
    swait(bufs[3], ss[3])
    swait(bufs[0], ss[0])


def _write_partial(acc_sh, part_hbm, cid, sid):
    for i in range(APT // AZC):
        rows = pl.ds(sid * APT + i * AZC, AZC)
        pltpu.sync_copy(acc_sh.at[rows], part_hbm.at[cid, rows])


@functools.partial(
    pl.kernel,
    out_type=jax.ShapeDtypeStruct((NSC, SCE), jnp.float32),
    mesh=_mesh,
    compiler_params=_params,
    scratch_types=[
        pltpu.VMEM((SCE,), jnp.int32),
        pltpu.VMEM((SCE,), jnp.int32),
        pltpu.VMEM((SCE,), jnp.float32),
        pltpu.VMEM((NPAD,), jnp.float32),
        pltpu.VMEM((SPT,), jnp.float32),
    ],
)
def _g_kernel(hf_hbm, tf_hbm, degp_hbm, g_hbm, h1s, t1s, g_v, dis_v, dtmp):
    cid = lax.axis_index("c")
    sid = lax.axis_index("s")
    wid = sid * NC + cid

    pltpu.sync_copy(degp_hbm.at[0], dis_v)
    for p in range(NPAD // SPT):
        pltpu.sync_copy(degp_hbm.at[1, pl.ds(p * SPT, SPT)], dtmp)

        def disbody(i, c):
            sl = pl.ds(p * SPT + i * G16, G16)
            d = dis_v[sl] + dtmp[pl.ds(i * G16, G16)]
            r = _rsqrt16(jnp.maximum(d, 1.0))
            dis_v[sl] = jnp.where(d > 0.0, r, 0.0)
            return c

        lax.fori_loop(0, SPT // G16, disbody, 0)

    for s in range(NSUP):
        sc = wid * NSUP + s
        pltpu.sync_copy(hf_hbm.at[sc], h1s)
        pltpu.sync_copy(tf_hbm.at[sc], t1s)

        def gbody(i, c):
            sl = pl.ds(i * G16, G16)
            gh = plsc.load_gather(dis_v, [h1s[sl]])
            gt = plsc.load_gather(dis_v, [t1s[sl]])
            g_v[sl] = gh * gt
            return c

        lax.fori_loop(0, SCE // G16, gbody, 0)
        pltpu.sync_copy(g_v, g_hbm.at[sc])


@functools.partial(
    pl.kernel,
    out_type=jax.ShapeDtypeStruct((NC, NPAD, D), jnp.float32),
    mesh=_mesh,
    compiler_params=_params,
    scratch_types=[
        pltpu.VMEM((SCH, CH), jnp.int32),
        pltpu.VMEM((SCE,), jnp.int32),
        pltpu.VMEM((SCE,), jnp.float32),
        pltpu.VMEM((CH, D), jnp.float32),
        pltpu.VMEM((CH, D), jnp.float32),
        pltpu.VMEM((CH, D), jnp.float32),
        pltpu.VMEM((CH, D), jnp.float32),
        pltpu.VMEM_SHARED((NPAD, D), jnp.float32),
        pltpu.SemaphoreType.DMA,
        pltpu.SemaphoreType.DMA,
        pltpu.SemaphoreType.DMA,
        pltpu.SemaphoreType.DMA,
        pltpu.SemaphoreType.DMA,
        pltpu.SemaphoreType.DMA,
        pltpu.SemaphoreType.DMA,
        pltpu.SemaphoreType.DMA,
    ],
)
def _layer_kernel(x_hbm, h3_hbm, tf_hbm, g_hbm, part_hbm,
                  h3s, t1s, g_v, buf0, buf1, buf2, buf3, acc_sh,
                  gs0, gs1, gs2, gs3, ss0, ss1, ss2, ss3):
    cid = lax.axis_index("c")
    sid = lax.axis_index("s")
    wid = sid * NC + cid
    _zero_acc(buf0, acc_sh, sid)
    plsc.subcore_barrier()
    for s in range(NSUP):
        sc = wid * NSUP + s
        pltpu.sync_copy(h3_hbm.at[sc], h3s)
        pltpu.sync_copy(tf_hbm.at[sc], t1s)
        pltpu.sync_copy(g_hbm.at[sc], g_v)
        _spmm_super(x_hbm, h3s, t1s, g_v, (buf0, buf1, buf2, buf3), acc_sh,
                    (gs0, gs1, gs2, gs3), (ss0, ss1, ss2, ss3))
    plsc.subcore_barrier()
    _write_partial(acc_sh, part_hbm, cid, sid)


CBR = 400


def _combine1_body(part_ref, x0_ref, out1_ref, emb1_ref):
    o1 = part_ref[0] + part_ref[1]
    out1_ref[...] = o1
    emb1_ref[...] = o1 + x0_ref[...]


def _combine1_kernel(part, x0):
    return pl.pallas_call(
        _combine1_body,
        grid=(N // CBR,),
        in_specs=[
            pl.BlockSpec((NC, CBR, D), lambda i: (0, i, 0)),
            pl.BlockSpec((CBR, D), lambda i: (i, 0)),
        ],
        out_specs=[
            pl.BlockSpec((CBR, D), lambda i: (i, 0)),
            pl.BlockSpec((CBR, D), lambda i: (i, 0)),
        ],
        out_shape=[
            jax.ShapeDtypeStruct((N, D), jnp.float32),
            jax.ShapeDtypeStruct((N, D), jnp.float32),
        ],
    )(part, x0)


def _combine2_body(part_ref, x0_ref, emb1_ref, out2_ref, summed_ref):
    o2 = part_ref[0] + part_ref[1]
    out2_ref[...] = o2
    summed_ref[...] = x0_ref[...] + 2.0 * emb1_ref[...] + o2


def _combine2_kernel(part, x0, emb1):
    return pl.pallas_call(
        _combine2_body,
        grid=(N // CBR,),
        in_specs=[
            pl.BlockSpec((NC, CBR, D), lambda i: (0, i, 0)),
            pl.BlockSpec((CBR, D), lambda i: (i, 0)),
            pl.BlockSpec((CBR, D), lambda i: (i, 0)),
        ],
        out_specs=[
            pl.BlockSpec((CBR, D), lambda i: (i, 0)),
            pl.BlockSpec((CBR, D), lambda i: (i, 0)),
        ],
        out_shape=[
            jax.ShapeDtypeStruct((N, D), jnp.float32),
            jax.ShapeDtypeStruct((N, D), jnp.float32),
        ],
    )(part, x0, emb1)


def kernel(user_emb, item_emb, h_list, t_list):
    x0 = jnp.concatenate([user_emb, item_emb], axis=0)
    h3 = h_list.reshape(NSC, SCH, CH)
    hf = h_list.reshape(NSC, SCE)
    tf = t_list.reshape(NSC, SCE)
    degp = _deg_kernel(h3)
    g = _g_kernel(hf, tf, degp)
    part1 = _layer_kernel(x0, h3, tf, g)
    out1, emb1 = _combine1_kernel(part1, x0)
    part2 = _layer_kernel(emb1, h3, tf, g)
    out2, summed = _combine2_kernel(part2, x0, emb1)
    return summed[:N_USERS], summed[N_USERS:], out1, out2


# speedup vs baseline: 19.6230x; 1.0305x over previous
import functools

import jax
import jax.numpy as jnp
from jax import lax
from jax.experimental import pallas as pl
from jax.experimental.pallas import tpu as pltpu
from jax.experimental.pallas import tpu_sc as plsc

N_USERS = 5000
N_ITEMS = 5000
N = N_USERS + N_ITEMS
E = 320000
D = 128
NC = 2
NS = 16
NW = NC * NS
EPW = E // NW
CH = 80
SCH = 25
SCE = SCH * CH
NSUP = EPW // SCE
NSC = NW * NSUP
NPAD = 10240
RPT = NPAD // NW
SPT = NPAD // NS
APT = NPAD // NS
AZC = 80
CR = 80
G16 = 16

_mesh = plsc.VectorSubcoreMesh(core_axis_name="c", subcore_axis_name="s")
_params = pltpu.CompilerParams(needs_layout_passes=False)


def _rsqrt16(x):
    i = lax.bitcast_convert_type(x, jnp.int32)
    i = jnp.int32(0x5F3759DF) - (i >> 1)
    y = lax.bitcast_convert_type(i, jnp.float32)
    for _ in range(3):
        y = y * (1.5 - 0.5 * x * y * y)
    return y


@functools.partial(
    pl.kernel,
    out_type=jax.ShapeDtypeStruct((NC, NPAD), jnp.float32),
    mesh=_mesh,
    compiler_params=_params,
    scratch_types=[
        pltpu.VMEM((SCH, CH), jnp.int32),
        pltpu.VMEM((CH,), jnp.float32),
        pltpu.VMEM((SPT,), jnp.float32),
        pltpu.VMEM_SHARED((NPAD,), jnp.float32),
        pltpu.SemaphoreType.DMA,
    ],
)
def _deg_kernel(h3_hbm, degp_hbm, h3s, ones_v, z_v, deg_sh, ssem):
    cid = lax.axis_index("c")
    sid = lax.axis_index("s")
    wid = sid * NC + cid

    def fill_ones(i, c):
        ones_v[pl.ds(i * G16, G16)] = jnp.full((G16,), 1.0, jnp.float32)
        return c

    lax.fori_loop(0, CH // G16, fill_ones, 0)

    def fill_zero(i, c):
        z_v[pl.ds(i * G16, G16)] = jnp.zeros((G16,), jnp.float32)
        return c

    lax.fori_loop(0, SPT // G16, fill_zero, 0)
    pltpu.sync_copy(z_v, deg_sh.at[pl.ds(sid * SPT, SPT)])
    plsc.subcore_barrier()

    for s in range(NSUP):
        pltpu.sync_copy(h3_hbm.at[wid * NSUP + s], h3s)

        def scat(j, c):
            pltpu.async_copy(ones_v, deg_sh.at[h3s.at[j]], ssem, add=True)
            return c

        lax.fori_loop(0, SCH, scat, 0)

        def drain(j, c):
            pltpu.make_async_copy(ones_v, deg_sh.at[h3s.at[0]], ssem).wait()
            return c

        lax.fori_loop(0, SCH, drain, 0)
    plsc.subcore_barrier()
    pltpu.sync_copy(deg_sh.at[pl.ds(sid * SPT, SPT)], z_v)
    pltpu.sync_copy(z_v, degp_hbm.at[cid, pl.ds(sid * SPT, SPT)])


def _zero_acc(buf, acc_sh, sid):
    def zrow(r, c):
        for k in range(D // G16):
            buf[r, pl.ds(k * G16, G16)] = jnp.zeros((G16,), jnp.float32)
        return c

    lax.fori_loop(0, CH, zrow, 0)
    for i in range(APT // AZC):
        pltpu.sync_copy(buf, acc_sh.at[pl.ds(sid * APT + i * AZC, AZC)])


def _scale_rows(buf, g_v, j):
    base = j * CH

    def blk(q, c):
        gvec = g_v[pl.ds(base + q * G16, G16)]
        for r16 in range(G16):
            gb = jnp.full((G16,), gvec[r16], jnp.float32)
            row = q * G16 + r16
            for k in range(D // G16):
                buf[row, pl.ds(k * G16, G16)] = buf[row, pl.ds(k * G16, G16)] * gb
        return c

    lax.fori_loop(0, CH // G16, blk, 0)


def _spmm_super(x_hbm, h3s, t1s, g_v, bufs, acc_sh, gs, ss):
    def gather(j, buf, sem):
        off = pl.multiple_of(j * CH, 16)
        return pltpu.async_copy(x_hbm.at[t1s.at[pl.ds(off, CH)]], buf, sem)

    def gwait(buf, sem):
        pltpu.make_async_copy(x_hbm.at[t1s.at[pl.ds(0, CH)]], buf, sem).wait()

    def scat(j, buf, sem):
        return pltpu.async_copy(buf, acc_sh.at[h3s.at[j]], sem, add=True)

    def swait(buf, sem):
        pltpu.make_async_copy(buf, acc_sh.at[h3s.at[0]], sem).wait()

    def step(j, b, with_swait, with_gather):
        gwait(bufs[b], gs[b])
        yb = (b + 2) % 4
        if with_swait:
            swait(bufs[yb], ss[yb])
        if with_gather:
            gather(j + 2, bufs[yb], gs[yb])
        _scale_rows(bufs[b], g_v, j)
        scat(j, bufs[b], ss[b])

    gather(0, bufs[0], gs[0])
    gather(1, bufs[1], gs[1])
    step(0, 0, False, True)
    step(1, 1, False, True)

    def quad(jj, c):
        j0 = jj * 4 + 2
        for i, b in enumerate((2, 3, 0, 1)):
            step(j0 + i, b, True, True)
        return c

    lax.fori_loop(0, (SCH - 5) // 4, quad, 0)
    step(SCH - 3, 2, True, True)
    step(SCH - 2, 3, True, False)
    step(SCH - 1, 0, True, False)
    swait(bufs[3], ss[3])
    swait(bufs[0], ss[0])


def _write_partial(acc_sh, part_hbm, cid, sid):
    for i in range(APT // AZC):
        rows = pl.ds(sid * APT + i * AZC, AZC)
        pltpu.sync_copy(acc_sh.at[rows], part_hbm.at[cid, rows])


@functools.partial(
    pl.kernel,
    out_type=jax.ShapeDtypeStruct((NSC, SCE), jnp.float32),
    mesh=_mesh,
    compiler_params=_params,
    scratch_types=[
        pltpu.VMEM((SCE,), jnp.int32),
        pltpu.VMEM((SCE,), jnp.int32),
        pltpu.VMEM((SCE,), jnp.int32),
        pltpu.VMEM((SCE,), jnp.int32),
        pltpu.VMEM((SCE,), jnp.float32),
        pltpu.VMEM((SCE,), jnp.float32),
        pltpu.VMEM((NPAD,), jnp.float32),
        pltpu.VMEM((SPT,), jnp.float32),
        pltpu.SemaphoreType.DMA,
        pltpu.SemaphoreType.DMA,
        pltpu.SemaphoreType.DMA,
        pltpu.SemaphoreType.DMA,
    ],
)
def _g_kernel(hf_hbm, tf_hbm, degp_hbm, g_hbm,
              h1a, h1b, t1a, t1b, g_va, g_vb, dis_v, dtmp,
              la, lb, wa, wb):
    cid = lax.axis_index("c")
    sid = lax.axis_index("s")
    wid = sid * NC + cid
    hs, ts, gbufs = (h1a, h1b), (t1a, t1b), (g_va, g_vb)
    ls, ws = (la, lb), (wa, wb)

    def loads(sch, b):
        sc = wid * NSUP + sch
        pltpu.async_copy(hf_hbm.at[sc], hs[b], ls[b])
        pltpu.async_copy(tf_hbm.at[sc], ts[b], ls[b])

    def loads_wait(b):
        pltpu.make_async_copy(hf_hbm.at[0], hs[b], ls[b]).wait()
        pltpu.make_async_copy(tf_hbm.at[0], ts[b], ls[b]).wait()

    loads(0, 0)
    loads(1, 1)

    pltpu.sync_copy(degp_hbm.at[0], dis_v)
    for p in range(NPAD // SPT):
        pltpu.sync_copy(degp_hbm.at[1, pl.ds(p * SPT, SPT)], dtmp)

        def disbody(i, c):
            sl = pl.ds(p * SPT + i * G16, G16)
            d = dis_v[sl] + dtmp[pl.ds(i * G16, G16)]
            r = _rsqrt16(jnp.maximum(d, 1.0))
            dis_v[sl] = jnp.where(d > 0.0, r, 0.0)
            return c

        lax.fori_loop(0, SPT // G16, disbody, 0)

    for sch in range(NSUP):
        b = sch % 2
        loads_wait(b)
        if sch >= 2:
            pltpu.make_async_copy(gbufs[b], g_hbm.at[0], ws[b]).wait()

        def gbody(i, c):
            sl = pl.ds(i * G16, G16)
            gh = plsc.load_gather(dis_v, [hs[b][sl]])
            gt = plsc.load_gather(dis_v, [ts[b][sl]])
            gbufs[b][sl] = gh * gt
            return c

        lax.fori_loop(0, SCE // G16, gbody, 0)
        if sch + 2 < NSUP:
            loads(sch + 2, b)
        pltpu.async_copy(gbufs[b], g_hbm.at[wid * NSUP + sch], ws[b])
    pltpu.make_async_copy(gbufs[1], g_hbm.at[0], ws[1]).wait()
    pltpu.make_async_copy(gbufs[0], g_hbm.at[0], ws[0]).wait()


@functools.partial(
    pl.kernel,
    out_type=jax.ShapeDtypeStruct((NC, NPAD, D), jnp.float32),
    mesh=_mesh,
    compiler_params=_params,
    scratch_types=[
        pltpu.VMEM((SCH, CH), jnp.int32),
        pltpu.VMEM((SCE,), jnp.int32),
        pltpu.VMEM((SCE,), jnp.float32),
        pltpu.VMEM((CH, D), jnp.float32),
        pltpu.VMEM((CH, D), jnp.float32),
        pltpu.VMEM((CH, D), jnp.float32),
        pltpu.VMEM((CH, D), jnp.float32),
        pltpu.VMEM_SHARED((NPAD, D), jnp.float32),
        pltpu.SemaphoreType.DMA,
        pltpu.SemaphoreType.DMA,
        pltpu.SemaphoreType.DMA,
        pltpu.SemaphoreType.DMA,
        pltpu.SemaphoreType.DMA,
        pltpu.SemaphoreType.DMA,
        pltpu.SemaphoreType.DMA,
        pltpu.SemaphoreType.DMA,
    ],
)
def _layer_kernel(x_hbm, h3_hbm, tf_hbm, g_hbm, part_hbm,
                  h3s, t1s, g_v, buf0, buf1, buf2, buf3, acc_sh,
                  gs0, gs1, gs2, gs3, ss0, ss1, ss2, ss3):
    cid = lax.axis_index("c")
    sid = lax.axis_index("s")
    wid = sid * NC + cid
    _zero_acc(buf0, acc_sh, sid)
    plsc.subcore_barrier()
    for s in range(NSUP):
        sc = wid * NSUP + s
        pltpu.sync_copy(h3_hbm.at[sc], h3s)
        pltpu.sync_copy(tf_hbm.at[sc], t1s)
        pltpu.sync_copy(g_hbm.at[sc], g_v)
        _spmm_super(x_hbm, h3s, t1s, g_v, (buf0, buf1, buf2, buf3), acc_sh,
                    (gs0, gs1, gs2, gs3), (ss0, ss1, ss2, ss3))
    plsc.subcore_barrier()
    _write_partial(acc_sh, part_hbm, cid, sid)


CBR = 400


def _combine1_body(part_ref, x0_ref, out1_ref, emb1_ref):
    o1 = part_ref[0] + part_ref[1]
    out1_ref[...] = o1
    emb1_ref[...] = o1 + x0_ref[...]


def _combine1_kernel(part, x0):
    return pl.pallas_call(
        _combine1_body,
        grid=(N // CBR,),
        in_specs=[
            pl.BlockSpec((NC, CBR, D), lambda i: (0, i, 0)),
            pl.BlockSpec((CBR, D), lambda i: (i, 0)),
        ],
        out_specs=[
            pl.BlockSpec((CBR, D), lambda i: (i, 0)),
            pl.BlockSpec((CBR, D), lambda i: (i, 0)),
        ],
        out_shape=[
            jax.ShapeDtypeStruct((N, D), jnp.float32),
            jax.ShapeDtypeStruct((N, D), jnp.float32),
        ],
    )(part, x0)


def _combine2_body(part_ref, x0_ref, emb1_ref, out2_ref, summed_ref):
    o2 = part_ref[0] + part_ref[1]
    out2_ref[...] = o2
    summed_ref[...] = x0_ref[...] + 2.0 * emb1_ref[...] + o2


def _combine2_kernel(part, x0, emb1):
    return pl.pallas_call(
        _combine2_body,
        grid=(N // CBR,),
        in_specs=[
            pl.BlockSpec((NC, CBR, D), lambda i: (0, i, 0)),
            pl.BlockSpec((CBR, D), lambda i: (i, 0)),
            pl.BlockSpec((CBR, D), lambda i: (i, 0)),
        ],
        out_specs=[
            pl.BlockSpec((CBR, D), lambda i: (i, 0)),
            pl.BlockSpec((CBR, D), lambda i: (i, 0)),
        ],
        out_shape=[
            jax.ShapeDtypeStruct((N, D), jnp.float32),
            jax.ShapeDtypeStruct((N, D), jnp.float32),
        ],
    )(part, x0, emb1)


def kernel(user_emb, item_emb, h_list, t_list):
    x0 = jnp.concatenate([user_emb, item_emb], axis=0)
    h3 = h_list.reshape(NSC, SCH, CH)
    hf = h_list.reshape(NSC, SCE)
    tf = t_list.reshape(NSC, SCE)
    degp = _deg_kernel(h3)
    g = _g_kernel(hf, tf, degp)
    part1 = _layer_kernel(x0, h3, tf, g)
    out1, emb1 = _combine1_kernel(part1, x0)
    part2 = _layer_kernel(emb1, h3, tf, g)
    out2, summed = _combine2_kernel(part2, x0, emb1)
    return summed[:N_USERS], summed[N_USERS:], out1, out2
